# baseline (speedup 1.0000x reference)
import jax
import jax.numpy as jnp
from jax.experimental import pallas as pl

L, H, D = 20, 128, 128
BN = 256


def _body(vi_ref, veh_ref, cust_ref, edge_ref, win_ref, bias_ref, wh_ref,
          mem_ref, out_ref):
    mem = mem_ref[...]
    vi = vi_ref[...]
    cur_h = jnp.zeros((BN, H), jnp.float32)
    for s in range(L):
        cur_h += jnp.where(vi == s, mem[s], 0.0)
    pre = jnp.dot(veh_ref[...], win_ref[0:D, :],
                  preferred_element_type=jnp.float32)
    pre += jnp.dot(cust_ref[...], win_ref[D:2 * D, :],
                   preferred_element_type=jnp.float32)
    pre += jnp.dot(edge_ref[...], win_ref[2 * D:3 * D, :],
                   preferred_element_type=jnp.float32)
    pre += jnp.dot(cur_h, wh_ref[...], preferred_element_type=jnp.float32)
    next_h = jnp.tanh(pre + bias_ref[...])
    for s in range(L):
        out_ref[s] = jnp.where(vi == s, next_h, mem[s])


@jax.jit
def kernel(memory, veh_idx, veh_repr, cust_repr, edge_emb, W_in, b_in,
           W_h, b_h):
    n, l, h = memory.shape
    grid = n // BN
    bias = (b_in + b_h).reshape(1, h)
    row = lambda i: (i, 0)
    slab = lambda i: (0, i, 0)
    full = lambda i: (0, 0)
    out = pl.pallas_call(
        _body,
        grid=(grid,),
        in_specs=[
            pl.BlockSpec((BN, 1), row),
            pl.BlockSpec((BN, D), row),
            pl.BlockSpec((BN, D), row),
            pl.BlockSpec((BN, D), row),
            pl.BlockSpec((3 * D, h), full),
            pl.BlockSpec((1, h), full),
            pl.BlockSpec((D, h), full),
            pl.BlockSpec((l, BN, h), slab),
        ],
        out_specs=pl.BlockSpec((l, BN, h), slab),
        out_shape=jax.ShapeDtypeStruct((l, n, h), memory.dtype),
    )(veh_idx, veh_repr[:, 0, :], cust_repr[:, 0, :], edge_emb[:, 0, 0, :],
      W_in, bias, W_h, memory.transpose(1, 0, 2))
    return out.transpose(1, 0, 2)


# speedup vs baseline: 1.1521x; 1.1521x over previous
import jax
import jax.numpy as jnp
from jax.experimental import pallas as pl

L, H, D = 20, 128, 128
BN = 1024


def _body(vi_ref, veh_ref, cust_ref, edge_ref, win_ref, bias_ref, wh_ref,
          mem_ref, out_ref):
    mem = mem_ref[...]
    vi = vi_ref[...]
    cur_h = jnp.zeros((BN, H), jnp.float32)
    for s in range(L):
        cur_h += jnp.where(vi == s, mem[s], 0.0)
    pre = jnp.dot(veh_ref[...], win_ref[0:D, :],
                  preferred_element_type=jnp.float32)
    pre += jnp.dot(cust_ref[...], win_ref[D:2 * D, :],
                   preferred_element_type=jnp.float32)
    pre += jnp.dot(edge_ref[...], win_ref[2 * D:3 * D, :],
                   preferred_element_type=jnp.float32)
    pre += jnp.dot(cur_h, wh_ref[...], preferred_element_type=jnp.float32)
    next_h = jnp.tanh(pre + bias_ref[...])
    for s in range(L):
        out_ref[s] = jnp.where(vi == s, next_h, mem[s])


@jax.jit
def kernel(memory, veh_idx, veh_repr, cust_repr, edge_emb, W_in, b_in,
           W_h, b_h):
    n, l, h = memory.shape
    grid = n // BN
    bias = (b_in + b_h).reshape(1, h)
    row = lambda i: (i, 0)
    slab = lambda i: (0, i, 0)
    full = lambda i: (0, 0)
    out = pl.pallas_call(
        _body,
        grid=(grid,),
        in_specs=[
            pl.BlockSpec((BN, 1), row),
            pl.BlockSpec((BN, D), row),
            pl.BlockSpec((BN, D), row),
            pl.BlockSpec((BN, D), row),
            pl.BlockSpec((3 * D, h), full),
            pl.BlockSpec((1, h), full),
            pl.BlockSpec((D, h), full),
            pl.BlockSpec((l, BN, h), slab),
        ],
        out_specs=pl.BlockSpec((l, BN, h), slab),
        out_shape=jax.ShapeDtypeStruct((l, n, h), memory.dtype),
    )(veh_idx, veh_repr[:, 0, :], cust_repr[:, 0, :], edge_emb[:, 0, 0, :],
      W_in, bias, W_h, memory.transpose(1, 0, 2))
    return out.transpose(1, 0, 2)
